# baseline (speedup 1.0000x reference)
import jax
import jax.numpy as jnp
from jax.experimental import pallas as pl

def _hardmax_block(x_ref, o_ref):
    big = jnp.int32(1 << 30)
    data = x_ref[0]
    rows = data.shape[0]
    r_iota = jax.lax.broadcasted_iota(jnp.int32, (rows, 128), 0)
    c_iota = jax.lax.broadcasted_iota(jnp.int32, (rows, 128), 1)
    flat = r_iota * 128 + c_iota

    mx = jnp.max(data, axis=0, keepdims=True)
    am = jnp.min(jnp.where(data == mx, flat, big), axis=0,
                 keepdims=True)

    val, idx = mx, am
    w = 128
    while w > 16:
        h = w // 2
        v1, v2 = val[:, :h], val[:, h:w]
        i1, i2 = idx[:, :h], idx[:, h:w]
        take2 = (v2 > v1) | ((v2 == v1) & (i2 < i1))
        val = jnp.where(take2, v2, v1)
        idx = jnp.where(take2, i2, i1)
        w = h

    for _ in range(3):
        idx = jnp.concatenate([idx, idx], axis=1)

    o_ref[0] = (flat == idx).astype(jnp.float32)


def kernel(x):
    b, n, m = x.shape
    rows = (n * m) // 128
    xf = x.reshape(b, rows, 128)
    out = pl.pallas_call(
        _hardmax_block,
        grid=(b,),
        in_specs=[pl.BlockSpec((1, rows, 128), lambda i: (i, 0, 0))],
        out_specs=pl.BlockSpec((1, rows, 128), lambda i: (i, 0, 0)),
        out_shape=jax.ShapeDtypeStruct((b, rows, 128), jnp.float32),
    )(xf)
    return out.reshape(b, n, m)
